# baseline (speedup 1.0000x reference)
import jax
import jax.numpy as jnp
from jax.experimental import pallas as pl
from jax.experimental.pallas import tpu as pltpu

_HINGE = 0.01
_K = 5
_N_SRC = 4096
_N_TGT = 4096
_D = 128
_LANES = 128
_BLK_C = 512
_CH = 32
_SCALE = 1.0 / (_N_SRC * _K)


def _body(s_ref, t_ref, o_ref, tt_ref):
    i = pl.program_id(0)
    ones_row = jnp.ones((1, _D), jnp.float32)

    @pl.when(i == 0)
    def _compute_tt():
        for c in range(_N_TGT // _BLK_C):
            tc = t_ref[c * _BLK_C:(c + 1) * _BLK_C, :]
            tt_ref[c * _BLK_C:(c + 1) * _BLK_C, :] = jax.lax.dot_general(
                tc * tc, ones_row, (((1,), (1,)), ((), ())),
                preferred_element_type=jnp.float32)

    s = s_ref[...]
    s2 = s * (-2.0)
    inf = jnp.float32(jnp.inf)
    msets = [[jnp.full((_CH, _LANES), inf, jnp.float32) for _ in range(_K)]
             for _ in range(4)]
    q_total = 0
    for c in range(_N_TGT // _BLK_C):
        tc = t_ref[c * _BLK_C:(c + 1) * _BLK_C, :]
        g = tt_ref[c * _BLK_C:(c + 1) * _BLK_C, :] + jax.lax.dot_general(
            tc, s2, (((1,), (1,)), ((), ())),
            preferred_element_type=jnp.float32)
        for q in range(_BLK_C // _CH):
            v = g[q * _CH:(q + 1) * _CH, :]
            m = msets[q_total % 4]
            q_total += 1
            for k in range(_K):
                lo = jnp.minimum(m[k], v)
                v = jnp.maximum(m[k], v)
                m[k] = lo

    def _keep5(a, b):
        return [jnp.minimum(a[k], b[_K - 1 - k]) for k in range(_K)]

    def _sort5(x):
        for a, b in ((0, 1), (3, 4), (2, 4), (2, 3), (0, 3),
                     (0, 2), (1, 4), (1, 3), (1, 2)):
            lo = jnp.minimum(x[a], x[b])
            hi = jnp.maximum(x[a], x[b])
            x[a], x[b] = lo, hi
        return x

    left = _sort5(_keep5(msets[0], msets[1]))
    right = _sort5(_keep5(msets[2], msets[3]))
    m = _keep5(left, right)

    ss = jax.lax.dot_general(
        ones_row, s * s, (((1,), (1,)), ((), ())),
        preferred_element_type=jnp.float32)
    cand = jnp.concatenate(m, axis=0)
    need = jnp.full((1, _LANES), float(_K), jnp.float32)
    acc = jnp.zeros((1, _LANES), jnp.float32)
    for _ in range(_K):
        mn = jnp.min(cand, axis=0, keepdims=True)
        eq = cand == mn
        cnt = jnp.sum(eq.astype(jnp.float32), axis=0, keepdims=True)
        take = jnp.minimum(cnt, need)
        d = jnp.sqrt(jnp.maximum(mn + ss, 1e-12))
        val = jnp.maximum(d - _HINGE, 0.0)
        val = jnp.where(take > 0, val, 0.0)
        acc = acc + take * val
        need = need - take
        cand = jnp.where(eq, inf, cand)
    total = jnp.sum(acc) * _SCALE

    @pl.when(i == 0)
    def _init_out():
        o_ref[0, 0] = 0.0

    o_ref[0, 0] += total


@jax.jit
def _run(source, target):
    out = pl.pallas_call(
        _body,
        grid=(_N_SRC // _LANES,),
        in_specs=[
            pl.BlockSpec((_LANES, _D), lambda i: (i, 0)),
            pl.BlockSpec((_N_TGT, _D), lambda i: (0, 0)),
        ],
        out_specs=pl.BlockSpec(memory_space=pltpu.SMEM),
        out_shape=jax.ShapeDtypeStruct((1, 1), jnp.float32),
        scratch_shapes=[pltpu.VMEM((_N_TGT, 1), jnp.float32)],
    )(source, target)
    return out[0, 0]


def kernel(source, target, top_k):
    loss = _run(source, target)
    return loss + 0.0 * jnp.asarray(top_k, dtype=loss.dtype)


# speedup vs baseline: 1.2942x; 1.2942x over previous
import jax
import jax.numpy as jnp
from jax.experimental import pallas as pl
from jax.experimental.pallas import tpu as pltpu

_HINGE = 0.01
_K = 5
_N_SRC = 4096
_N_TGT = 4096
_D = 128
_SRC_BLK = 256
_BLK_C = 512
_CH = 32
_SCALE = 1.0 / (_N_SRC * _K)


def _keep5(a, b):
    return [jnp.minimum(a[k], b[_K - 1 - k]) for k in range(_K)]


def _sort5(x):
    for a, b in ((0, 1), (3, 4), (2, 4), (2, 3), (0, 3),
                 (0, 2), (1, 4), (1, 3), (1, 2)):
        lo = jnp.minimum(x[a], x[b])
        hi = jnp.maximum(x[a], x[b])
        x[a], x[b] = lo, hi
    return x


def _body(s_ref, t_ref, o_ref, tt_ref):
    i = pl.program_id(0)
    ones_row = jnp.ones((1, _D), jnp.float32)

    @pl.when(i == 0)
    def _compute_tt():
        for c in range(_N_TGT // _BLK_C):
            tc = t_ref[c * _BLK_C:(c + 1) * _BLK_C, :]
            tt_ref[c * _BLK_C:(c + 1) * _BLK_C, :] = jax.lax.dot_general(
                tc * tc, ones_row, (((1,), (1,)), ((), ())),
                preferred_element_type=jnp.float32)

    s = s_ref[...]
    s2 = s * (-2.0)
    inf = jnp.float32(jnp.inf)
    m = [jnp.full((_CH, _SRC_BLK), inf, jnp.float32) for _ in range(_K)]
    for c in range(_N_TGT // _BLK_C):
        tc = t_ref[c * _BLK_C:(c + 1) * _BLK_C, :]
        g = tt_ref[c * _BLK_C:(c + 1) * _BLK_C, :] + jax.lax.dot_general(
            tc, s2, (((1,), (1,)), ((), ())),
            preferred_element_type=jnp.float32)
        for q in range(_BLK_C // _CH):
            v = g[q * _CH:(q + 1) * _CH, :]
            for k in range(_K):
                lo = jnp.minimum(m[k], v)
                v = jnp.maximum(m[k], v)
                m[k] = lo

    h = _CH
    while h > 1:
        h //= 2
        m = _keep5([x[:h] for x in m], [x[h:] for x in m])
        if h > 1:
            m = _sort5(m)

    ss = jax.lax.dot_general(
        ones_row, s * s, (((1,), (1,)), ((), ())),
        preferred_element_type=jnp.float32)
    acc = jnp.zeros((1, _SRC_BLK), jnp.float32)
    for k in range(_K):
        d = jnp.sqrt(jnp.maximum(m[k] + ss, 1e-12))
        acc = acc + jnp.maximum(d - _HINGE, 0.0)
    total = jnp.sum(acc) * _SCALE

    @pl.when(i == 0)
    def _init_out():
        o_ref[0, 0] = 0.0

    o_ref[0, 0] += total


@jax.jit
def _run(source, target):
    out = pl.pallas_call(
        _body,
        grid=(_N_SRC // _SRC_BLK,),
        in_specs=[
            pl.BlockSpec((_SRC_BLK, _D), lambda i: (i, 0)),
            pl.BlockSpec((_N_TGT, _D), lambda i: (0, 0)),
        ],
        out_specs=pl.BlockSpec(memory_space=pltpu.SMEM),
        out_shape=jax.ShapeDtypeStruct((1, 1), jnp.float32),
        scratch_shapes=[pltpu.VMEM((_N_TGT, 1), jnp.float32)],
    )(source, target)
    return out[0, 0]


def kernel(source, target, top_k):
    loss = _run(source, target)
    return loss + 0.0 * jnp.asarray(top_k, dtype=loss.dtype)


# speedup vs baseline: 1.5581x; 1.2038x over previous
import jax
import jax.numpy as jnp
from jax.experimental import pallas as pl
from jax.experimental.pallas import tpu as pltpu

_HINGE = 0.01
_K = 5
_N_SRC = 4096
_N_TGT = 4096
_D = 128
_SRC_BLK = 256
_BLK_C = 512
_CH = 32
_SCALE = 1.0 / (_N_SRC * _K)


def _keep5(a, b):
    return [jnp.minimum(a[k], b[_K - 1 - k]) for k in range(_K)]


def _sort5(x):
    for a, b in ((0, 1), (3, 4), (2, 4), (2, 3), (0, 3),
                 (0, 2), (1, 4), (1, 3), (1, 2)):
        lo = jnp.minimum(x[a], x[b])
        hi = jnp.maximum(x[a], x[b])
        x[a], x[b] = lo, hi
    return x


def _body(s_ref, t_ref, o_ref, tt_ref):
    i = pl.program_id(0)
    ones_row = jnp.ones((1, _D), jnp.float32)

    @pl.when(i == 0)
    def _compute_tt():
        for c in range(_N_TGT // _BLK_C):
            tc = t_ref[c * _BLK_C:(c + 1) * _BLK_C, :]
            tt_ref[c * _BLK_C:(c + 1) * _BLK_C, :] = jax.lax.dot_general(
                tc * tc, ones_row, (((1,), (1,)), ((), ())),
                preferred_element_type=jnp.float32)

    s = s_ref[...]
    s2 = s * (-2.0)
    inf = jnp.float32(jnp.inf)

    def _ins2(t, x):
        lo = jnp.minimum(t[0], x)
        hi = jnp.maximum(t[0], x)
        return [lo, jnp.minimum(t[1], hi)]

    m = [jnp.full((_CH, _SRC_BLK), inf, jnp.float32) for _ in range(_K)]
    mp = [jnp.full((_CH, _SRC_BLK), inf, jnp.float32) for _ in range(2)]
    mq = [jnp.full((_CH, _SRC_BLK), inf, jnp.float32) for _ in range(2)]
    mr = [jnp.full((_CH, _SRC_BLK), inf, jnp.float32) for _ in range(2)]
    for c in range(_N_TGT // _BLK_C):
        tc = t_ref[c * _BLK_C:(c + 1) * _BLK_C, :]
        g = tt_ref[c * _BLK_C:(c + 1) * _BLK_C, :] + jax.lax.dot_general(
            tc, s2, (((1,), (1,)), ((), ())),
            preferred_element_type=jnp.float32)
        vs = [g[q * _CH:(q + 1) * _CH, :] for q in range(_BLK_C // _CH)]
        for lvl, track in ((0, mp), (1, mq), (2, mr)):
            nxt = []
            for j in range(len(vs) // 2):
                a, b = vs[2 * j], vs[2 * j + 1]
                nxt.append(jnp.minimum(a, b))
                w = jnp.maximum(a, b)
                track[:] = _ins2(track, w)
            vs = nxt
        for v in vs:
            for k in range(_K):
                lo = jnp.minimum(m[k], v)
                v = jnp.maximum(m[k], v)
                m[k] = lo

    t4 = [mp[0], mp[1], mq[0], mq[1]]
    for a, b in ((0, 2), (1, 3), (1, 2)):
        lo = jnp.minimum(t4[a], t4[b])
        hi = jnp.maximum(t4[a], t4[b])
        t4[a], t4[b] = lo, hi
    m = _sort5([m[0], jnp.minimum(m[1], t4[3]), jnp.minimum(m[2], t4[2]),
                jnp.minimum(m[3], t4[1]), jnp.minimum(m[4], t4[0])])
    m = _sort5([m[0], m[1], m[2], jnp.minimum(m[3], mr[1]),
                jnp.minimum(m[4], mr[0])])

    h = _CH
    while h > 1:
        h //= 2
        m = _keep5([x[:h] for x in m], [x[h:] for x in m])
        if h > 1:
            m = _sort5(m)

    ss = jax.lax.dot_general(
        ones_row, s * s, (((1,), (1,)), ((), ())),
        preferred_element_type=jnp.float32)
    acc = jnp.zeros((1, _SRC_BLK), jnp.float32)
    for k in range(_K):
        d = jnp.sqrt(jnp.maximum(m[k] + ss, 1e-12))
        acc = acc + jnp.maximum(d - _HINGE, 0.0)
    total = jnp.sum(acc) * _SCALE

    @pl.when(i == 0)
    def _init_out():
        o_ref[0, 0] = 0.0

    o_ref[0, 0] += total


@jax.jit
def _run(source, target):
    out = pl.pallas_call(
        _body,
        grid=(_N_SRC // _SRC_BLK,),
        in_specs=[
            pl.BlockSpec((_SRC_BLK, _D), lambda i: (i, 0)),
            pl.BlockSpec((_N_TGT, _D), lambda i: (0, 0)),
        ],
        out_specs=pl.BlockSpec(memory_space=pltpu.SMEM),
        out_shape=jax.ShapeDtypeStruct((1, 1), jnp.float32),
        scratch_shapes=[pltpu.VMEM((_N_TGT, 1), jnp.float32)],
    )(source, target)
    return out[0, 0]


def kernel(source, target, top_k):
    loss = _run(source, target)
    return loss + 0.0 * jnp.asarray(top_k, dtype=loss.dtype)


# speedup vs baseline: 1.7686x; 1.1351x over previous
import jax
import jax.numpy as jnp
from jax.experimental import pallas as pl
from jax.experimental.pallas import tpu as pltpu

_HINGE = 0.01
_K = 5
_N_SRC = 4096
_N_TGT = 4096
_D = 128
_SRC_BLK = 1024
_BLK_C = 512
_CH = 32
_SCALE = 1.0 / (_N_SRC * _K)


def _keep5(a, b):
    return [jnp.minimum(a[k], b[_K - 1 - k]) for k in range(_K)]


def _sort5(x):
    for a, b in ((0, 1), (3, 4), (2, 4), (2, 3), (0, 3),
                 (0, 2), (1, 4), (1, 3), (1, 2)):
        lo = jnp.minimum(x[a], x[b])
        hi = jnp.maximum(x[a], x[b])
        x[a], x[b] = lo, hi
    return x


def _body(s_ref, t_ref, o_ref, tt_ref):
    i = pl.program_id(0)
    ones_row = jnp.ones((1, _D), jnp.float32)

    @pl.when(i == 0)
    def _compute_tt():
        for c in range(_N_TGT // _BLK_C):
            tc = t_ref[c * _BLK_C:(c + 1) * _BLK_C, :]
            tt_ref[c * _BLK_C:(c + 1) * _BLK_C, :] = jax.lax.dot_general(
                tc * tc, ones_row, (((1,), (1,)), ((), ())),
                preferred_element_type=jnp.float32)

    s = s_ref[...]
    s2 = s * (-2.0)
    inf = jnp.float32(jnp.inf)

    def _ins2(t, x):
        lo = jnp.minimum(t[0], x)
        hi = jnp.maximum(t[0], x)
        return [lo, jnp.minimum(t[1], hi)]

    m = [jnp.full((_CH, _SRC_BLK), inf, jnp.float32) for _ in range(_K)]
    mp = [jnp.full((_CH, _SRC_BLK), inf, jnp.float32) for _ in range(2)]
    mq = [jnp.full((_CH, _SRC_BLK), inf, jnp.float32) for _ in range(2)]
    mr = [jnp.full((_CH, _SRC_BLK), inf, jnp.float32) for _ in range(2)]
    for c in range(_N_TGT // _BLK_C):
        tc = t_ref[c * _BLK_C:(c + 1) * _BLK_C, :]
        g = tt_ref[c * _BLK_C:(c + 1) * _BLK_C, :] + jax.lax.dot_general(
            tc, s2, (((1,), (1,)), ((), ())),
            preferred_element_type=jnp.float32)
        vs = [g[q * _CH:(q + 1) * _CH, :] for q in range(_BLK_C // _CH)]
        for lvl, track in ((0, mp), (1, mq), (2, mr)):
            nxt = []
            for j in range(len(vs) // 2):
                a, b = vs[2 * j], vs[2 * j + 1]
                nxt.append(jnp.minimum(a, b))
                w = jnp.maximum(a, b)
                track[:] = _ins2(track, w)
            vs = nxt
        for v in vs:
            for k in range(_K):
                lo = jnp.minimum(m[k], v)
                v = jnp.maximum(m[k], v)
                m[k] = lo

    t4 = [mp[0], mp[1], mq[0], mq[1]]
    for a, b in ((0, 2), (1, 3), (1, 2)):
        lo = jnp.minimum(t4[a], t4[b])
        hi = jnp.maximum(t4[a], t4[b])
        t4[a], t4[b] = lo, hi
    m = _sort5([m[0], jnp.minimum(m[1], t4[3]), jnp.minimum(m[2], t4[2]),
                jnp.minimum(m[3], t4[1]), jnp.minimum(m[4], t4[0])])
    m = _sort5([m[0], m[1], m[2], jnp.minimum(m[3], mr[1]),
                jnp.minimum(m[4], mr[0])])

    h = _CH
    while h > 1:
        h //= 2
        m = _keep5([x[:h] for x in m], [x[h:] for x in m])
        if h > 1:
            m = _sort5(m)

    ss = jax.lax.dot_general(
        ones_row, s * s, (((1,), (1,)), ((), ())),
        preferred_element_type=jnp.float32)
    acc = jnp.zeros((1, _SRC_BLK), jnp.float32)
    for k in range(_K):
        d = jnp.sqrt(jnp.maximum(m[k] + ss, 1e-12))
        acc = acc + jnp.maximum(d - _HINGE, 0.0)
    total = jnp.sum(acc) * _SCALE

    @pl.when(i == 0)
    def _init_out():
        o_ref[0, 0] = 0.0

    o_ref[0, 0] += total


@jax.jit
def _run(source, target):
    out = pl.pallas_call(
        _body,
        grid=(_N_SRC // _SRC_BLK,),
        in_specs=[
            pl.BlockSpec((_SRC_BLK, _D), lambda i: (i, 0)),
            pl.BlockSpec((_N_TGT, _D), lambda i: (0, 0)),
        ],
        out_specs=pl.BlockSpec(memory_space=pltpu.SMEM),
        out_shape=jax.ShapeDtypeStruct((1, 1), jnp.float32),
        scratch_shapes=[pltpu.VMEM((_N_TGT, 1), jnp.float32)],
    )(source, target)
    return out[0, 0]


def kernel(source, target, top_k):
    loss = _run(source, target)
    return loss + 0.0 * jnp.asarray(top_k, dtype=loss.dtype)


# speedup vs baseline: 1.8115x; 1.0242x over previous
import jax
import jax.numpy as jnp
from jax.experimental import pallas as pl
from jax.experimental.pallas import tpu as pltpu

_HINGE = 0.01
_K = 5
_N_SRC = 4096
_N_TGT = 4096
_D = 128
_SRC_BLK = 2048
_BLK_C = 512
_CH = 32
_SCALE = 1.0 / (_N_SRC * _K)


def _keep5(a, b):
    return [jnp.minimum(a[k], b[_K - 1 - k]) for k in range(_K)]


def _sort5(x):
    for a, b in ((0, 1), (3, 4), (2, 4), (2, 3), (0, 3),
                 (0, 2), (1, 4), (1, 3), (1, 2)):
        lo = jnp.minimum(x[a], x[b])
        hi = jnp.maximum(x[a], x[b])
        x[a], x[b] = lo, hi
    return x


def _body(s_ref, t_ref, o_ref, tt_ref):
    i = pl.program_id(0)
    ones_row = jnp.ones((1, _D), jnp.float32)

    @pl.when(i == 0)
    def _compute_tt():
        for c in range(_N_TGT // _BLK_C):
            tc = t_ref[c * _BLK_C:(c + 1) * _BLK_C, :]
            tt_ref[c * _BLK_C:(c + 1) * _BLK_C, :] = jax.lax.dot_general(
                tc * tc, ones_row, (((1,), (1,)), ((), ())),
                preferred_element_type=jnp.float32)

    s = s_ref[...]
    s2 = s * (-2.0)
    inf = jnp.float32(jnp.inf)

    def _ins2(t, x):
        lo = jnp.minimum(t[0], x)
        hi = jnp.maximum(t[0], x)
        return [lo, jnp.minimum(t[1], hi)]

    m = [jnp.full((_CH, _SRC_BLK), inf, jnp.float32) for _ in range(_K)]
    mp = [jnp.full((_CH, _SRC_BLK), inf, jnp.float32) for _ in range(2)]
    mq = [jnp.full((_CH, _SRC_BLK), inf, jnp.float32) for _ in range(2)]
    mr = [jnp.full((_CH, _SRC_BLK), inf, jnp.float32) for _ in range(2)]
    for c in range(_N_TGT // _BLK_C):
        tc = t_ref[c * _BLK_C:(c + 1) * _BLK_C, :]
        g = tt_ref[c * _BLK_C:(c + 1) * _BLK_C, :] + jax.lax.dot_general(
            tc, s2, (((1,), (1,)), ((), ())),
            preferred_element_type=jnp.float32)
        vs = [g[q * _CH:(q + 1) * _CH, :] for q in range(_BLK_C // _CH)]
        for lvl, track in ((0, mp), (1, mq), (2, mr)):
            nxt = []
            for j in range(len(vs) // 2):
                a, b = vs[2 * j], vs[2 * j + 1]
                nxt.append(jnp.minimum(a, b))
                w = jnp.maximum(a, b)
                track[:] = _ins2(track, w)
            vs = nxt
        for v in vs:
            for k in range(_K):
                lo = jnp.minimum(m[k], v)
                v = jnp.maximum(m[k], v)
                m[k] = lo

    t4 = [mp[0], mp[1], mq[0], mq[1]]
    for a, b in ((0, 2), (1, 3), (1, 2)):
        lo = jnp.minimum(t4[a], t4[b])
        hi = jnp.maximum(t4[a], t4[b])
        t4[a], t4[b] = lo, hi
    m = _sort5([m[0], jnp.minimum(m[1], t4[3]), jnp.minimum(m[2], t4[2]),
                jnp.minimum(m[3], t4[1]), jnp.minimum(m[4], t4[0])])
    m = _sort5([m[0], m[1], m[2], jnp.minimum(m[3], mr[1]),
                jnp.minimum(m[4], mr[0])])

    h = _CH
    while h > 1:
        h //= 2
        m = _keep5([x[:h] for x in m], [x[h:] for x in m])
        if h > 1:
            m = _sort5(m)

    ss = jax.lax.dot_general(
        ones_row, s * s, (((1,), (1,)), ((), ())),
        preferred_element_type=jnp.float32)
    acc = jnp.zeros((1, _SRC_BLK), jnp.float32)
    for k in range(_K):
        d = jnp.sqrt(jnp.maximum(m[k] + ss, 1e-12))
        acc = acc + jnp.maximum(d - _HINGE, 0.0)
    total = jnp.sum(acc) * _SCALE

    @pl.when(i == 0)
    def _init_out():
        o_ref[0, 0] = 0.0

    o_ref[0, 0] += total


@jax.jit
def _run(source, target):
    out = pl.pallas_call(
        _body,
        grid=(_N_SRC // _SRC_BLK,),
        in_specs=[
            pl.BlockSpec((_SRC_BLK, _D), lambda i: (i, 0)),
            pl.BlockSpec((_N_TGT, _D), lambda i: (0, 0)),
        ],
        out_specs=pl.BlockSpec(memory_space=pltpu.SMEM),
        out_shape=jax.ShapeDtypeStruct((1, 1), jnp.float32),
        scratch_shapes=[pltpu.VMEM((_N_TGT, 1), jnp.float32)],
    )(source, target)
    return out[0, 0]


def kernel(source, target, top_k):
    loss = _run(source, target)
    return loss + 0.0 * jnp.asarray(top_k, dtype=loss.dtype)


# speedup vs baseline: 2.1022x; 1.1605x over previous
import jax
import jax.numpy as jnp
from jax.experimental import pallas as pl
from jax.experimental.pallas import tpu as pltpu

_HINGE = 0.01
_K = 5
_N_SRC = 4096
_N_TGT = 4096
_D = 128
_SRC_BLK = 2048
_BLK_C = 512
_CH = 32
_DA = 136
_SCALE = 1.0 / (_N_SRC * _K)


def _keep5(a, b):
    return [jnp.minimum(a[k], b[_K - 1 - k]) for k in range(_K)]


def _sort5(x):
    for a, b in ((0, 1), (3, 4), (2, 4), (2, 3), (0, 3),
                 (0, 2), (1, 4), (1, 3), (1, 2)):
        lo = jnp.minimum(x[a], x[b])
        hi = jnp.maximum(x[a], x[b])
        x[a], x[b] = lo, hi
    return x


def _body(s_ref, t_ref, o_ref, ta_ref):
    i = pl.program_id(0)
    ones_row = jnp.ones((1, _D), jnp.float32)

    @pl.when(i == 0)
    def _augment_t():
        for c in range(_N_TGT // _BLK_C):
            tc = t_ref[c * _BLK_C:(c + 1) * _BLK_C, :]
            ta_ref[c * _BLK_C:(c + 1) * _BLK_C, :_D] = tc
            ta_ref[c * _BLK_C:(c + 1) * _BLK_C, _D:_D + 1] = jax.lax.dot_general(
                tc * tc, ones_row, (((1,), (1,)), ((), ())),
                preferred_element_type=jnp.float32)
            ta_ref[c * _BLK_C:(c + 1) * _BLK_C, _D + 1:] = jnp.zeros(
                (_BLK_C, _DA - _D - 1), jnp.float32)

    s = s_ref[...]
    s2 = jnp.concatenate(
        [s * (-2.0), jnp.ones((_SRC_BLK, 1), jnp.float32),
         jnp.zeros((_SRC_BLK, _DA - _D - 1), jnp.float32)],
        axis=1)
    inf = jnp.float32(jnp.inf)

    def _ins2(t, x):
        lo = jnp.minimum(t[0], x)
        hi = jnp.maximum(t[0], x)
        return [lo, jnp.minimum(t[1], hi)]

    m = [jnp.full((_CH, _SRC_BLK), inf, jnp.float32) for _ in range(_K)]
    mp = [jnp.full((_CH, _SRC_BLK), inf, jnp.float32) for _ in range(2)]
    mq = [jnp.full((_CH, _SRC_BLK), inf, jnp.float32) for _ in range(2)]
    mr = [jnp.full((_CH, _SRC_BLK), inf, jnp.float32) for _ in range(2)]
    mt = [jnp.full((_CH, _SRC_BLK), inf, jnp.float32) for _ in range(2)]
    for c in range(_N_TGT // _BLK_C):
        g = jax.lax.dot_general(
            ta_ref[c * _BLK_C:(c + 1) * _BLK_C, :], s2,
            (((1,), (1,)), ((), ())),
            preferred_element_type=jnp.float32)
        vs = [g[q * _CH:(q + 1) * _CH, :] for q in range(_BLK_C // _CH)]
        for lvl, track in ((0, mp), (1, mq), (2, mr), (3, mt)):
            nxt = []
            for j in range(len(vs) // 2):
                a, b = vs[2 * j], vs[2 * j + 1]
                nxt.append(jnp.minimum(a, b))
                w = jnp.maximum(a, b)
                track[:] = _ins2(track, w)
            vs = nxt
        for v in vs:
            for k in range(_K):
                lo = jnp.minimum(m[k], v)
                v = jnp.maximum(m[k], v)
                m[k] = lo

    def _merge22(x):
        for a, b in ((0, 2), (1, 3), (1, 2)):
            lo = jnp.minimum(x[a], x[b])
            hi = jnp.maximum(x[a], x[b])
            x[a], x[b] = lo, hi
        return x

    for t4 in (_merge22([mp[0], mp[1], mq[0], mq[1]]),
               _merge22([mr[0], mr[1], mt[0], mt[1]])):
        m = _sort5([m[0], jnp.minimum(m[1], t4[3]), jnp.minimum(m[2], t4[2]),
                    jnp.minimum(m[3], t4[1]), jnp.minimum(m[4], t4[0])])

    h = _CH
    while h > 1:
        h //= 2
        m = _keep5([x[:h] for x in m], [x[h:] for x in m])
        if h > 1:
            m = _sort5(m)

    ss = jax.lax.dot_general(
        ones_row, s * s, (((1,), (1,)), ((), ())),
        preferred_element_type=jnp.float32)
    acc = jnp.zeros((1, _SRC_BLK), jnp.float32)
    for k in range(_K):
        d = jnp.sqrt(jnp.maximum(m[k] + ss, 1e-12))
        acc = acc + jnp.maximum(d - _HINGE, 0.0)
    total = jnp.sum(acc) * _SCALE

    @pl.when(i == 0)
    def _init_out():
        o_ref[0, 0] = 0.0

    o_ref[0, 0] += total


@jax.jit
def _run(source, target):
    out = pl.pallas_call(
        _body,
        grid=(_N_SRC // _SRC_BLK,),
        in_specs=[
            pl.BlockSpec((_SRC_BLK, _D), lambda i: (i, 0)),
            pl.BlockSpec((_N_TGT, _D), lambda i: (0, 0)),
        ],
        out_specs=pl.BlockSpec(memory_space=pltpu.SMEM),
        out_shape=jax.ShapeDtypeStruct((1, 1), jnp.float32),
        scratch_shapes=[pltpu.VMEM((_N_TGT, _DA), jnp.float32)],
    )(source, target)
    return out[0, 0]


def kernel(source, target, top_k):
    loss = _run(source, target)
    return loss + 0.0 * jnp.asarray(top_k, dtype=loss.dtype)


# speedup vs baseline: 2.3809x; 1.1326x over previous
import jax
import jax.numpy as jnp
from jax.experimental import pallas as pl
from jax.experimental.pallas import tpu as pltpu

_HINGE = 0.01
_K = 5
_N_SRC = 4096
_N_TGT = 4096
_D = 128
_SRC_BLK = 2048
_BLK_C = 512
_CH = 8
_DA = 136
_SCALE = 1.0 / (_N_SRC * _K)


def _keep5(a, b):
    return [jnp.minimum(a[k], b[_K - 1 - k]) for k in range(_K)]


def _sort5(x):
    for a, b in ((0, 1), (3, 4), (2, 4), (2, 3), (0, 3),
                 (0, 2), (1, 4), (1, 3), (1, 2)):
        lo = jnp.minimum(x[a], x[b])
        hi = jnp.maximum(x[a], x[b])
        x[a], x[b] = lo, hi
    return x


def _body(s_ref, t_ref, o_ref, ta_ref):
    i = pl.program_id(0)
    ones_row = jnp.ones((1, _D), jnp.float32)

    @pl.when(i == 0)
    def _augment_t():
        for c in range(_N_TGT // _BLK_C):
            tc = t_ref[c * _BLK_C:(c + 1) * _BLK_C, :]
            ta_ref[c * _BLK_C:(c + 1) * _BLK_C, :_D] = tc
            ta_ref[c * _BLK_C:(c + 1) * _BLK_C, _D:_D + 1] = jax.lax.dot_general(
                tc * tc, ones_row, (((1,), (1,)), ((), ())),
                preferred_element_type=jnp.float32)
            ta_ref[c * _BLK_C:(c + 1) * _BLK_C, _D + 1:] = jnp.zeros(
                (_BLK_C, _DA - _D - 1), jnp.float32)

    s = s_ref[...]
    s2 = jnp.concatenate(
        [s * (-2.0), jnp.ones((_SRC_BLK, 1), jnp.float32),
         jnp.zeros((_SRC_BLK, _DA - _D - 1), jnp.float32)],
        axis=1)
    inf = jnp.float32(jnp.inf)

    def _ins2(t, x):
        lo = jnp.minimum(t[0], x)
        hi = jnp.maximum(t[0], x)
        return [lo, jnp.minimum(t[1], hi)]

    m = [jnp.full((_CH, _SRC_BLK), inf, jnp.float32) for _ in range(_K)]
    mp = [jnp.full((_CH, _SRC_BLK), inf, jnp.float32) for _ in range(2)]
    mq = [jnp.full((_CH, _SRC_BLK), inf, jnp.float32) for _ in range(2)]
    mr = [jnp.full((_CH, _SRC_BLK), inf, jnp.float32) for _ in range(2)]
    mt = [jnp.full((_CH, _SRC_BLK), inf, jnp.float32) for _ in range(2)]
    for c in range(_N_TGT // _BLK_C):
        g = jax.lax.dot_general(
            ta_ref[c * _BLK_C:(c + 1) * _BLK_C, :], s2,
            (((1,), (1,)), ((), ())),
            preferred_element_type=jnp.float32)
        vs = [g[q * _CH:(q + 1) * _CH, :] for q in range(_BLK_C // _CH)]
        for lvl, track in ((0, mp), (1, mq), (2, mr), (3, mt)):
            nxt = []
            for j in range(len(vs) // 2):
                a, b = vs[2 * j], vs[2 * j + 1]
                nxt.append(jnp.minimum(a, b))
                w = jnp.maximum(a, b)
                track[:] = _ins2(track, w)
            vs = nxt
        for v in vs:
            for k in range(_K):
                lo = jnp.minimum(m[k], v)
                v = jnp.maximum(m[k], v)
                m[k] = lo

    def _merge22(x):
        for a, b in ((0, 2), (1, 3), (1, 2)):
            lo = jnp.minimum(x[a], x[b])
            hi = jnp.maximum(x[a], x[b])
            x[a], x[b] = lo, hi
        return x

    for t4 in (_merge22([mp[0], mp[1], mq[0], mq[1]]),
               _merge22([mr[0], mr[1], mt[0], mt[1]])):
        m = _sort5([m[0], jnp.minimum(m[1], t4[3]), jnp.minimum(m[2], t4[2]),
                    jnp.minimum(m[3], t4[1]), jnp.minimum(m[4], t4[0])])

    h = _CH
    while h > 1:
        h //= 2
        m = _keep5([x[:h] for x in m], [x[h:] for x in m])
        if h > 1:
            m = _sort5(m)

    ss = jax.lax.dot_general(
        ones_row, s * s, (((1,), (1,)), ((), ())),
        preferred_element_type=jnp.float32)
    acc = jnp.zeros((1, _SRC_BLK), jnp.float32)
    for k in range(_K):
        d = jnp.sqrt(jnp.maximum(m[k] + ss, 1e-12))
        acc = acc + jnp.maximum(d - _HINGE, 0.0)
    total = jnp.sum(acc) * _SCALE

    @pl.when(i == 0)
    def _init_out():
        o_ref[0, 0] = 0.0

    o_ref[0, 0] += total


@jax.jit
def _run(source, target):
    out = pl.pallas_call(
        _body,
        grid=(_N_SRC // _SRC_BLK,),
        in_specs=[
            pl.BlockSpec((_SRC_BLK, _D), lambda i: (i, 0)),
            pl.BlockSpec((_N_TGT, _D), lambda i: (0, 0)),
        ],
        out_specs=pl.BlockSpec(memory_space=pltpu.SMEM),
        out_shape=jax.ShapeDtypeStruct((1, 1), jnp.float32),
        scratch_shapes=[pltpu.VMEM((_N_TGT, _DA), jnp.float32)],
    )(source, target)
    return out[0, 0]


def kernel(source, target, top_k):
    loss = _run(source, target)
    return loss + 0.0 * jnp.asarray(top_k, dtype=loss.dtype)


# speedup vs baseline: 2.4285x; 1.0200x over previous
import jax
import jax.numpy as jnp
from jax.experimental import pallas as pl
from jax.experimental.pallas import tpu as pltpu

_HINGE = 0.01
_K = 5
_N_SRC = 4096
_N_TGT = 4096
_D = 128
_SRC_BLK = 4096
_BLK_C = 512
_CH = 8
_DA = 136
_SCALE = 1.0 / (_N_SRC * _K)


def _keep5(a, b):
    return [jnp.minimum(a[k], b[_K - 1 - k]) for k in range(_K)]


def _sort5(x):
    for a, b in ((0, 1), (3, 4), (2, 4), (2, 3), (0, 3),
                 (0, 2), (1, 4), (1, 3), (1, 2)):
        lo = jnp.minimum(x[a], x[b])
        hi = jnp.maximum(x[a], x[b])
        x[a], x[b] = lo, hi
    return x


def _body(s_ref, t_ref, o_ref, ta_ref):
    i = pl.program_id(0)
    ones_row = jnp.ones((1, _D), jnp.float32)

    @pl.when(i == 0)
    def _augment_t():
        for c in range(_N_TGT // _BLK_C):
            tc = t_ref[c * _BLK_C:(c + 1) * _BLK_C, :]
            ta_ref[c * _BLK_C:(c + 1) * _BLK_C, :_D] = tc
            ta_ref[c * _BLK_C:(c + 1) * _BLK_C, _D:_D + 1] = jax.lax.dot_general(
                tc * tc, ones_row, (((1,), (1,)), ((), ())),
                preferred_element_type=jnp.float32)
            ta_ref[c * _BLK_C:(c + 1) * _BLK_C, _D + 1:] = jnp.zeros(
                (_BLK_C, _DA - _D - 1), jnp.float32)

    s = s_ref[...]
    s2 = jnp.concatenate(
        [s * (-2.0), jnp.ones((_SRC_BLK, 1), jnp.float32),
         jnp.zeros((_SRC_BLK, _DA - _D - 1), jnp.float32)],
        axis=1)
    inf = jnp.float32(jnp.inf)

    def _ins2(t, x):
        lo = jnp.minimum(t[0], x)
        hi = jnp.maximum(t[0], x)
        return [lo, jnp.minimum(t[1], hi)]

    m = [jnp.full((_CH, _SRC_BLK), inf, jnp.float32) for _ in range(_K)]
    mp = [jnp.full((_CH, _SRC_BLK), inf, jnp.float32) for _ in range(2)]
    mq = [jnp.full((_CH, _SRC_BLK), inf, jnp.float32) for _ in range(2)]
    mr = [jnp.full((_CH, _SRC_BLK), inf, jnp.float32) for _ in range(2)]
    mt = [jnp.full((_CH, _SRC_BLK), inf, jnp.float32) for _ in range(2)]
    for c in range(_N_TGT // _BLK_C):
        g = jax.lax.dot_general(
            ta_ref[c * _BLK_C:(c + 1) * _BLK_C, :], s2,
            (((1,), (1,)), ((), ())),
            preferred_element_type=jnp.float32)
        vs = [g[q * _CH:(q + 1) * _CH, :] for q in range(_BLK_C // _CH)]
        for lvl, track in ((0, mp), (1, mq), (2, mr), (3, mt)):
            nxt = []
            for j in range(len(vs) // 2):
                a, b = vs[2 * j], vs[2 * j + 1]
                nxt.append(jnp.minimum(a, b))
                w = jnp.maximum(a, b)
                track[:] = _ins2(track, w)
            vs = nxt
        for v in vs:
            for k in range(_K):
                lo = jnp.minimum(m[k], v)
                v = jnp.maximum(m[k], v)
                m[k] = lo

    def _merge22(x):
        for a, b in ((0, 2), (1, 3), (1, 2)):
            lo = jnp.minimum(x[a], x[b])
            hi = jnp.maximum(x[a], x[b])
            x[a], x[b] = lo, hi
        return x

    for t4 in (_merge22([mp[0], mp[1], mq[0], mq[1]]),
               _merge22([mr[0], mr[1], mt[0], mt[1]])):
        m = _sort5([m[0], jnp.minimum(m[1], t4[3]), jnp.minimum(m[2], t4[2]),
                    jnp.minimum(m[3], t4[1]), jnp.minimum(m[4], t4[0])])

    h = _CH
    while h > 1:
        h //= 2
        m = _keep5([x[:h] for x in m], [x[h:] for x in m])
        if h > 1:
            m = _sort5(m)

    ss = jax.lax.dot_general(
        ones_row, s * s, (((1,), (1,)), ((), ())),
        preferred_element_type=jnp.float32)
    acc = jnp.zeros((1, _SRC_BLK), jnp.float32)
    for k in range(_K):
        d = jnp.sqrt(jnp.maximum(m[k] + ss, 1e-12))
        acc = acc + jnp.maximum(d - _HINGE, 0.0)
    total = jnp.sum(acc) * _SCALE

    @pl.when(i == 0)
    def _init_out():
        o_ref[0, 0] = 0.0

    o_ref[0, 0] += total


@jax.jit
def _run(source, target):
    out = pl.pallas_call(
        _body,
        grid=(_N_SRC // _SRC_BLK,),
        in_specs=[
            pl.BlockSpec((_SRC_BLK, _D), lambda i: (i, 0)),
            pl.BlockSpec((_N_TGT, _D), lambda i: (0, 0)),
        ],
        out_specs=pl.BlockSpec(memory_space=pltpu.SMEM),
        out_shape=jax.ShapeDtypeStruct((1, 1), jnp.float32),
        scratch_shapes=[pltpu.VMEM((_N_TGT, _DA), jnp.float32)],
    )(source, target)
    return out[0, 0]


def kernel(source, target, top_k):
    loss = _run(source, target)
    return loss + 0.0 * jnp.asarray(top_k, dtype=loss.dtype)
